# baseline (speedup 1.0000x reference)
import jax
import jax.numpy as jnp
from jax import lax
from jax.experimental import pallas as pl
from jax.experimental.pallas import tpu as pltpu
from jax.experimental.pallas import tpu_sc as plsc

VOCAB = 50265
HIDDEN = 1024
PROMPT_LENGTH = 10
BATCH = 4
SEQ = 2048

NC, NS = 2, 16
NW = NC * NS
ROWS = BATCH * SEQ
RPW = ROWS // NW
CH = 32
NCH = RPW // CH


NBUF = 3


def _body(tok_hbm, table_hbm, prompts_hbm, out_hbm,
          tok_v, idx_v, buf0, buf1, buf2, pbuf, g0, g1, g2, s0, s1, s2):
    wid = lax.axis_index("s") * NC + lax.axis_index("c")
    base = wid * RPW
    pltpu.sync_copy(tok_hbm, tok_v)
    b = wid // (NW // BATCH)
    col0 = (wid % (NW // BATCH)) * RPW
    for k in range(RPW // 16):
        idx_v[pl.ds(k * 16, 16)] = tok_v[b, pl.ds(col0 + k * 16, 16)]

    bufs = (buf0, buf1, buf2)
    gsem = (g0, g1, g2)
    ssem = (s0, s1, s2)

    def gather(c):
        return pltpu.make_async_copy(
            table_hbm.at[idx_v.at[pl.ds(c * CH, CH)]],
            bufs[c % NBUF], gsem[c % NBUF])

    def scatter(c):
        return pltpu.make_async_copy(
            bufs[c % NBUF], out_hbm.at[pl.ds(base + c * CH, CH)],
            ssem[c % NBUF])

    gather(0).start()
    gather(1).start()
    for c in range(NCH):
        gather(c).wait()
        if c == 0:
            @pl.when(wid % (NW // BATCH) == 0)
            def _():
                pltpu.sync_copy(prompts_hbm.at[pl.ds(0, 8)],
                                bufs[0].at[pl.ds(0, 8)])
                pltpu.sync_copy(prompts_hbm, pbuf)
                for r in range(8, PROMPT_LENGTH):
                    for k in range(HIDDEN // 16):
                        bufs[0][r, pl.ds(k * 16, 16)] = \
                            pbuf[r, pl.ds(k * 16, 16)]
        scatter(c).start()
        if c + 2 < NCH:
            if c >= 1:
                scatter(c - 1).wait()
            gather(c + 2).start()
    for c in range(max(0, NCH - 3), NCH):
        scatter(c).wait()


def kernel(tokens, wte_weight, prompts):
    run = pl.kernel(
        _body,
        out_type=jax.ShapeDtypeStruct((ROWS, HIDDEN), jnp.float32),
        mesh=plsc.VectorSubcoreMesh(core_axis_name="c", subcore_axis_name="s"),
        scratch_types=[
            pltpu.VMEM((BATCH, SEQ), jnp.int32),
            pltpu.VMEM((RPW,), jnp.int32),
            pltpu.VMEM((CH, HIDDEN), jnp.float32),
            pltpu.VMEM((CH, HIDDEN), jnp.float32),
            pltpu.VMEM((CH, HIDDEN), jnp.float32),
            pltpu.VMEM((PROMPT_LENGTH, HIDDEN), jnp.float32),
            pltpu.SemaphoreType.DMA,
            pltpu.SemaphoreType.DMA,
            pltpu.SemaphoreType.DMA,
            pltpu.SemaphoreType.DMA,
            pltpu.SemaphoreType.DMA,
            pltpu.SemaphoreType.DMA,
        ],
    )
    out = run(tokens, wte_weight, prompts)
    return out.reshape(BATCH, SEQ, HIDDEN)


# speedup vs baseline: 1.0419x; 1.0419x over previous
import jax
import jax.numpy as jnp
from jax import lax
from jax.experimental import pallas as pl
from jax.experimental.pallas import tpu as pltpu
from jax.experimental.pallas import tpu_sc as plsc

VOCAB = 50265
HIDDEN = 1024
PROMPT_LENGTH = 10
BATCH = 4
SEQ = 2048

NC, NS = 2, 16
NW = NC * NS
ROWS = BATCH * SEQ
RPW = ROWS // NW
CHUNKS = (48, 48, 48, 48, 48, 16)
NCH = len(CHUNKS)
OFFS = (0, 48, 96, 144, 192, 240)


NBUF = 2


def _body(tok_hbm, table_hbm, prompts_hbm, out_hbm,
          idx_v, buf0, buf1, pbuf, g0, g1, s0, s1):
    wid = lax.axis_index("s") * NC + lax.axis_index("c")
    base = wid * RPW
    pltpu.sync_copy(tok_hbm.at[pl.ds(base, RPW)], idx_v)

    bufs = (buf0, buf1)
    gsem = (g0, g1)
    ssem = (s0, s1)

    def gather(c):
        return pltpu.make_async_copy(
            table_hbm.at[idx_v.at[pl.ds(OFFS[c], CHUNKS[c])]],
            bufs[c % NBUF].at[pl.ds(0, CHUNKS[c])], gsem[c % NBUF])

    def scatter(c):
        return pltpu.make_async_copy(
            bufs[c % NBUF].at[pl.ds(0, CHUNKS[c])],
            out_hbm.at[pl.ds(base + OFFS[c], CHUNKS[c])],
            ssem[c % NBUF])

    gather(0).start()
    for c in range(NCH):
        if c + 1 < NCH:
            if c >= 1:
                scatter(c - 1).wait()
            gather(c + 1).start()
        gather(c).wait()
        if c == 0:
            @pl.when(wid % (NW // BATCH) == 0)
            def _():
                pltpu.sync_copy(prompts_hbm.at[pl.ds(0, 8)],
                                bufs[0].at[pl.ds(0, 8)])
                pltpu.sync_copy(prompts_hbm, pbuf)
                for r in range(8, PROMPT_LENGTH):
                    for k in range(HIDDEN // 16):
                        bufs[0][r, pl.ds(k * 16, 16)] = \
                            pbuf[r, pl.ds(k * 16, 16)]
        scatter(c).start()
    scatter(NCH - 2).wait()
    scatter(NCH - 1).wait()


def kernel(tokens, wte_weight, prompts):
    tok_flat = tokens.reshape(ROWS)
    run = pl.kernel(
        _body,
        out_type=jax.ShapeDtypeStruct((ROWS, HIDDEN), jnp.float32),
        mesh=plsc.VectorSubcoreMesh(core_axis_name="c", subcore_axis_name="s"),
        scratch_types=[
            pltpu.VMEM((RPW,), jnp.int32),
            pltpu.VMEM((CHUNKS[0], HIDDEN), jnp.float32),
            pltpu.VMEM((CHUNKS[0], HIDDEN), jnp.float32),
            pltpu.VMEM((PROMPT_LENGTH, HIDDEN), jnp.float32),
            pltpu.SemaphoreType.DMA,
            pltpu.SemaphoreType.DMA,
            pltpu.SemaphoreType.DMA,
            pltpu.SemaphoreType.DMA,
        ],
    )
    out = run(tok_flat, wte_weight, prompts)
    return out.reshape(BATCH, SEQ, HIDDEN)


# speedup vs baseline: 1.0432x; 1.0012x over previous
import jax
import jax.numpy as jnp
from jax import lax
from jax.experimental import pallas as pl
from jax.experimental.pallas import tpu as pltpu
from jax.experimental.pallas import tpu_sc as plsc

VOCAB = 50265
HIDDEN = 1024
PROMPT_LENGTH = 10
BATCH = 4
SEQ = 2048

NC, NS = 2, 16
NW = NC * NS
ROWS = BATCH * SEQ
RPW = ROWS // NW
CH = 32
NCH = RPW // CH


NBUF = 3


def _body(tok_hbm, table_hbm, prompts_hbm, out_hbm,
          tok_v, idx_v, buf0, buf1, buf2, pbuf, g0, g1, g2, s0, s1, s2):
    wid = lax.axis_index("s") * NC + lax.axis_index("c")
    base = wid * RPW
    b = wid // (NW // BATCH)
    col0 = (wid % (NW // BATCH)) * RPW
    pltpu.sync_copy(tok_hbm.at[:, pl.ds(col0, RPW)], tok_v)
    for k in range(RPW // 16):
        idx_v[pl.ds(k * 16, 16)] = tok_v[b, pl.ds(k * 16, 16)]

    bufs = (buf0, buf1, buf2)
    gsem = (g0, g1, g2)
    ssem = (s0, s1, s2)

    def gather(c):
        return pltpu.make_async_copy(
            table_hbm.at[idx_v.at[pl.ds(c * CH, CH)]],
            bufs[c % NBUF], gsem[c % NBUF])

    def scatter(c):
        return pltpu.make_async_copy(
            bufs[c % NBUF], out_hbm.at[pl.ds(base + c * CH, CH)],
            ssem[c % NBUF])

    gather(0).start()
    gather(1).start()
    for c in range(NCH):
        gather(c).wait()
        if c == 0:
            @pl.when(wid % (NW // BATCH) == 0)
            def _():
                pltpu.sync_copy(prompts_hbm.at[pl.ds(0, 8)],
                                bufs[0].at[pl.ds(0, 8)])
                pltpu.sync_copy(prompts_hbm, pbuf)
                for r in range(8, PROMPT_LENGTH):
                    for k in range(HIDDEN // 16):
                        bufs[0][r, pl.ds(k * 16, 16)] = \
                            pbuf[r, pl.ds(k * 16, 16)]
        scatter(c).start()
        if c + 2 < NCH:
            if c >= 1:
                scatter(c - 1).wait()
            gather(c + 2).start()
    for c in range(max(0, NCH - 3), NCH):
        scatter(c).wait()


def kernel(tokens, wte_weight, prompts):
    run = pl.kernel(
        _body,
        out_type=jax.ShapeDtypeStruct((ROWS, HIDDEN), jnp.float32),
        mesh=plsc.VectorSubcoreMesh(core_axis_name="c", subcore_axis_name="s"),
        scratch_types=[
            pltpu.VMEM((BATCH, RPW), jnp.int32),
            pltpu.VMEM((RPW,), jnp.int32),
            pltpu.VMEM((CH, HIDDEN), jnp.float32),
            pltpu.VMEM((CH, HIDDEN), jnp.float32),
            pltpu.VMEM((CH, HIDDEN), jnp.float32),
            pltpu.VMEM((PROMPT_LENGTH, HIDDEN), jnp.float32),
            pltpu.SemaphoreType.DMA,
            pltpu.SemaphoreType.DMA,
            pltpu.SemaphoreType.DMA,
            pltpu.SemaphoreType.DMA,
            pltpu.SemaphoreType.DMA,
            pltpu.SemaphoreType.DMA,
        ],
    )
    out = run(tokens, wte_weight, prompts)
    return out.reshape(BATCH, SEQ, HIDDEN)


# speedup vs baseline: 1.0496x; 1.0062x over previous
import jax
import jax.numpy as jnp
from jax import lax
from jax.experimental import pallas as pl
from jax.experimental.pallas import tpu as pltpu
from jax.experimental.pallas import tpu_sc as plsc

VOCAB = 50265
HIDDEN = 1024
PROMPT_LENGTH = 10
BATCH = 4
SEQ = 2048

NC, NS = 2, 16
NW = NC * NS
ROWS = BATCH * SEQ
RPW = ROWS // NW
CH = 32
NCH = RPW // CH


NBUF = 3


def _body(tok_hbm, table_hbm, prompts_hbm, out_hbm,
          idx_v, buf0, buf1, buf2, pbuf, g0, g1, g2, s0, s1, s2):
    wid = lax.axis_index("s") * NC + lax.axis_index("c")
    base = wid * RPW
    pltpu.sync_copy(tok_hbm.at[pl.ds(base, RPW)], idx_v)

    bufs = (buf0, buf1, buf2)
    gsem = (g0, g1, g2)
    ssem = (s0, s1, s2)

    def gather(c):
        return pltpu.make_async_copy(
            table_hbm.at[idx_v.at[pl.ds(c * CH, CH)]],
            bufs[c % NBUF], gsem[c % NBUF])

    def scatter(c):
        return pltpu.make_async_copy(
            bufs[c % NBUF], out_hbm.at[pl.ds(base + c * CH, CH)],
            ssem[c % NBUF])

    gather(0).start()
    gather(1).start()
    for c in range(NCH):
        gather(c).wait()
        if c == 0:
            @pl.when(wid % (NW // BATCH) == 0)
            def _():
                pltpu.sync_copy(prompts_hbm.at[pl.ds(0, 8)],
                                bufs[0].at[pl.ds(0, 8)])
                pltpu.sync_copy(prompts_hbm, pbuf)
                for r in range(8, PROMPT_LENGTH):
                    for k in range(HIDDEN // 16):
                        bufs[0][r, pl.ds(k * 16, 16)] = \
                            pbuf[r, pl.ds(k * 16, 16)]
        scatter(c).start()
        if c + 2 < NCH:
            if c >= 1:
                scatter(c - 1).wait()
            gather(c + 2).start()
    for c in range(max(0, NCH - 3), NCH):
        scatter(c).wait()


def kernel(tokens, wte_weight, prompts):
    tok_flat = tokens.reshape(ROWS)
    run = pl.kernel(
        _body,
        out_type=jax.ShapeDtypeStruct((ROWS, HIDDEN), jnp.float32),
        mesh=plsc.VectorSubcoreMesh(core_axis_name="c", subcore_axis_name="s"),
        scratch_types=[
            pltpu.VMEM((RPW,), jnp.int32),
            pltpu.VMEM((CH, HIDDEN), jnp.float32),
            pltpu.VMEM((CH, HIDDEN), jnp.float32),
            pltpu.VMEM((CH, HIDDEN), jnp.float32),
            pltpu.VMEM((PROMPT_LENGTH, HIDDEN), jnp.float32),
            pltpu.SemaphoreType.DMA,
            pltpu.SemaphoreType.DMA,
            pltpu.SemaphoreType.DMA,
            pltpu.SemaphoreType.DMA,
            pltpu.SemaphoreType.DMA,
            pltpu.SemaphoreType.DMA,
        ],
    )
    out = run(tok_flat, wte_weight, prompts)
    return out.reshape(BATCH, SEQ, HIDDEN)


# speedup vs baseline: 1.0621x; 1.0119x over previous
import jax
import jax.numpy as jnp
from jax import lax
from jax.experimental import pallas as pl
from jax.experimental.pallas import tpu as pltpu
from jax.experimental.pallas import tpu_sc as plsc

VOCAB = 50265
HIDDEN = 1024
PROMPT_LENGTH = 10
BATCH = 4
SEQ = 2048

NC, NS = 2, 16
NW = NC * NS
ROWS = BATCH * SEQ
RPW = ROWS // NW
CH = 32
NCH = RPW // CH


PROMPT_PAD = 16
NBUF = 3


def _body(tok_hbm, table_hbm, prompts_hbm, out_hbm,
          idx_v, buf0, buf1, buf2, pbuf, g0, g1, g2, s0, s1, s2):
    wid = lax.axis_index("s") * NC + lax.axis_index("c")
    base = wid * RPW
    pltpu.sync_copy(tok_hbm.at[pl.ds(base, RPW)], idx_v)

    bufs = (buf0, buf1, buf2)
    gsem = (g0, g1, g2)
    ssem = (s0, s1, s2)

    def gather(c):
        return pltpu.make_async_copy(
            table_hbm.at[idx_v.at[pl.ds(c * CH, CH)]],
            bufs[c % NBUF], gsem[c % NBUF])

    def scatter(c):
        return pltpu.make_async_copy(
            bufs[c % NBUF], out_hbm.at[pl.ds(base + c * CH, CH)],
            ssem[c % NBUF])

    gather(0).start()
    gather(1).start()
    for c in range(NCH):
        gather(c).wait()
        if c == 0:
            @pl.when(wid % (NW // BATCH) == 0)
            def _():
                pltpu.sync_copy(prompts_hbm.at[pl.ds(0, 8)],
                                bufs[0].at[pl.ds(0, 8)])
                pltpu.sync_copy(prompts_hbm, pbuf)
                for r in range(8, PROMPT_LENGTH):
                    for k in range(HIDDEN // 16):
                        bufs[0][r, pl.ds(k * 16, 16)] = \
                            pbuf[r, pl.ds(k * 16, 16)]
        scatter(c).start()
        if c + 2 < NCH:
            if c >= 1:
                scatter(c - 1).wait()
            gather(c + 2).start()
    for c in range(max(0, NCH - 3), NCH):
        scatter(c).wait()


def kernel(tokens, wte_weight, prompts):
    tok_flat = tokens.reshape(ROWS)
    prompts_pad = jnp.concatenate(
        [prompts, jnp.zeros((PROMPT_PAD - PROMPT_LENGTH, HIDDEN),
                            prompts.dtype)], axis=0)
    run = pl.kernel(
        _body,
        out_type=jax.ShapeDtypeStruct((ROWS, HIDDEN), jnp.float32),
        mesh=plsc.VectorSubcoreMesh(core_axis_name="c", subcore_axis_name="s"),
        scratch_types=[
            pltpu.VMEM((RPW,), jnp.int32),
            pltpu.VMEM((CH, HIDDEN), jnp.float32),
            pltpu.VMEM((CH, HIDDEN), jnp.float32),
            pltpu.VMEM((CH, HIDDEN), jnp.float32),
            pltpu.VMEM((PROMPT_PAD, HIDDEN), jnp.float32),
            pltpu.SemaphoreType.DMA,
            pltpu.SemaphoreType.DMA,
            pltpu.SemaphoreType.DMA,
            pltpu.SemaphoreType.DMA,
            pltpu.SemaphoreType.DMA,
            pltpu.SemaphoreType.DMA,
        ],
    )
    out = run(tok_flat, wte_weight, prompts_pad)
    return out.reshape(BATCH, SEQ, HIDDEN)
